# Initial kernel scaffold; baseline (speedup 1.0000x reference)
import jax
import jax.numpy as jnp
from jax.experimental import pallas as pl


def kernel(x, z_degree, edge_index):
    raise NotImplementedError("write your pallas kernel here")


# speedup vs baseline: 1.1609x; 1.1609x over previous
import functools

import jax
import jax.numpy as jnp
from jax import lax
from jax.experimental import pallas as pl
from jax.experimental.pallas import tpu as pltpu
from jax.experimental.pallas import tpu_sc as plsc

MAXDEG = 8
DIM = 256
P1 = 5
PF = 4
NROWS = 300
NPAD = 304
ROW_BLK = 8


def _idiv(a, n):
    return lax.div(a, jnp.full((16,), n, jnp.int32))


def _sc_gather_tile(edge_index, z_degree):
    info = plsc.get_sparse_core_info()
    nc = info.num_cores
    mesh = plsc.VectorSubcoreMesh(core_axis_name="c", subcore_axis_name="s")

    @functools.partial(
        pl.kernel,
        mesh=mesh,
        compiler_params=pltpu.CompilerParams(needs_layout_passes=False),
        out_type=jax.ShapeDtypeStruct((NPAD, DIM), jnp.float32),
        scratch_types=[
            pltpu.VMEM((48,), jnp.int32),
            pltpu.VMEM((32,), jnp.int32),
            pltpu.VMEM((128,), jnp.int32),
            pltpu.VMEM((128,), jnp.int32),
            pltpu.VMEM((48,), jnp.int32),
            pltpu.VMEM((128, DIM), jnp.float32),
            pltpu.VMEM((128, DIM), jnp.float32),
            pltpu.VMEM((48, DIM), jnp.float32),
            pltpu.SemaphoreType.DMA,
        ],
    )
    def sc_k(edge_hbm, z_hbm, out_hbm, src_v, deg_v, ia_v, ib_v, ic_v,
             ra_v, rb_v, rc_v, sem):
        wid = lax.axis_index("s") * nc + lax.axis_index("c")

        @pl.when(wid == 0)
        def _():
            pltpu.sync_copy(edge_hbm.at[0], src_v)
            zero16 = jnp.zeros((16,), jnp.int32)
            deg_v[pl.ds(0, 16)] = zero16
            deg_v[pl.ds(16, 16)] = zero16
            one16 = jnp.ones((16,), jnp.int32)
            for e in range(3):
                plsc.addupdate_scatter(deg_v, [src_v[pl.ds(e * 16, 16)]], one16)
            for ch in range(2):
                d = deg_v[pl.ds(ch * 16, 16)]
                d = jnp.maximum(jnp.minimum(d, MAXDEG) - 1, 0)
                deg_v[pl.ds(ch * 16, 16)] = d
            idx_bufs = ((ia_v, 0), (ib_v, 128), (ic_v, 256))
            for buf, base in idx_bufs:
                for ch in range(buf.shape[0] // 16):
                    r = lax.iota(jnp.int32, 16) + (base + ch * 16)
                    wh = _idiv(r, 20)
                    p1 = _idiv(r - wh * 20, 4)
                    j = (wh - _idiv(wh, 5) * 5) * 5 + p1
                    buf[pl.ds(ch * 16, 16)] = plsc.load_gather(deg_v, [j])
            cps = [
                pltpu.async_copy(z_hbm.at[ia_v], ra_v, sem),
                pltpu.async_copy(z_hbm.at[ib_v], rb_v, sem),
                pltpu.async_copy(z_hbm.at[ic_v], rc_v, sem),
            ]
            for cp in cps:
                cp.wait()
            pltpu.sync_copy(ra_v, out_hbm.at[pl.ds(0, 128)])
            pltpu.sync_copy(rb_v, out_hbm.at[pl.ds(128, 128)])
            pltpu.sync_copy(rc_v, out_hbm.at[pl.ds(256, 48)])

    return sc_k(edge_index, z_degree)


def _tc_broadcast(tile, bf):
    def body(t_ref, o_ref):
        o_ref[...] = jnp.broadcast_to(t_ref[...][None, :, :],
                                      (ROW_BLK, 15, 5120))

    return pl.pallas_call(
        body,
        grid=(bf // ROW_BLK,),
        in_specs=[pl.BlockSpec((15, 5120), lambda i: (0, 0))],
        out_specs=pl.BlockSpec((ROW_BLK, 15, 5120), lambda i: (i, 0, 0)),
        out_shape=jax.ShapeDtypeStruct((bf, 15, 5120), jnp.float32),
    )(tile)


def kernel(x, z_degree, edge_index):
    b, _, F, J, Wc = x.shape
    f = F // PF
    rows = _sc_gather_tile(edge_index, z_degree)
    tile = rows[:NROWS].reshape(15, 5120)
    out = _tc_broadcast(tile, b * f)
    return out.reshape(b, f, 15, 5120)
